# baseline (speedup 1.0000x reference)
import functools

import jax
import jax.numpy as jnp
import numpy as np
from jax import lax
from jax.experimental import pallas as pl
from jax.experimental.pallas import tpu as pltpu
from jax.experimental.pallas import tpu_sc as plsc

N = 10000
E = 320000
C = 128
T = 7
NC, NS = 2, 16
NW = NC * NS
EPW = E // NW
CHUNK = 80
NCHUNK = EPW // CHUNK
NROWC = N // CHUNK
SCALE = 1.0 / (T * np.sqrt(float(C)))

_f32 = jnp.float32


_BN = 1000
_E2 = E // C


def _proj_body(x_ref, w_ref, ei_ref, typ_ref, y_ref, g_ref, r_ref):
    x = x_ref[...]
    for t in range(T):
        y_ref[t] = jnp.dot(x, w_ref[t], preferred_element_type=_f32) * SCALE

    @pl.when(pl.program_id(0) == 0)
    def _edge_lists():
        g_ref[...] = typ_ref[...] * N + ei_ref[1]
        r_ref[...] = ei_ref[0]


def _project(x, w3, ei3, typ2d):
    zero3 = lambda nb: (0, 0, 0)
    zero2 = lambda nb: (0, 0)
    return pl.pallas_call(
        _proj_body,
        grid=(N // _BN,),
        in_specs=[
            pl.BlockSpec((_BN, C), lambda nb: (nb, 0)),
            pl.BlockSpec((T, C, C), zero3),
            pl.BlockSpec((2, _E2, C), zero3),
            pl.BlockSpec((_E2, C), zero2),
        ],
        out_specs=[
            pl.BlockSpec((T, _BN, C), lambda nb: (0, nb, 0)),
            pl.BlockSpec((_E2, C), zero2),
            pl.BlockSpec((_E2, C), zero2),
        ],
        out_shape=(
            jax.ShapeDtypeStruct((T, N, C), _f32),
            jax.ShapeDtypeStruct((_E2, C), jnp.int32),
            jax.ShapeDtypeStruct((_E2, C), jnp.int32),
        ),
    )(x, w3, ei3, typ2d)


_mesh = plsc.VectorSubcoreMesh(core_axis_name="c", subcore_axis_name="s")


@functools.partial(
    pl.kernel,
    out_type=jax.ShapeDtypeStruct((NC, N, C), _f32),
    mesh=_mesh,
    scratch_types=[
        pltpu.VMEM((NCHUNK, CHUNK), jnp.int32),
        pltpu.VMEM((1, CHUNK), jnp.int32),
        pltpu.VMEM((1, CHUNK), jnp.int32),
        pltpu.VMEM((1, CHUNK), jnp.int32),
        pltpu.VMEM((CHUNK, C), _f32),
        pltpu.VMEM((CHUNK, C), _f32),
        pltpu.VMEM((CHUNK, C), _f32),
        pltpu.VMEM_SHARED((N, C), _f32),
        pltpu.SemaphoreType.DMA,
        pltpu.SemaphoreType.DMA,
        pltpu.SemaphoreType.DMA,
    ],
    compiler_params=pltpu.CompilerParams(use_tc_tiling_on_sc=False),
)
def _sc_aggregate(y_hbm, gidx_hbm, rows_hbm, out_hbm,
                  gidx_v, rb0, rb1, rb2, gb0, gb1, gb2, acc,
                  sem0, sem1, sem2):
    cid = lax.axis_index("c")
    sid = lax.axis_index("s")
    wid = sid * NC + cid
    rows_h = rows_hbm.at[wid]

    pltpu.sync_copy(gidx_hbm.at[wid], gidx_v)

    zeros16 = jnp.zeros((16,), _f32)

    def _zrow(i, carry):
        for j in range(C // 16):
            gb0[i, pl.ds(j * 16, 16)] = zeros16
        return carry

    lax.fori_loop(0, CHUNK, _zrow, 0)
    for i in range(pl.cdiv(NROWC, NS)):
        zc = i * NS + sid

        @pl.when(zc < NROWC)
        def _zero_chunk():
            off = pl.multiple_of(zc * CHUNK, CHUNK)
            pltpu.sync_copy(gb0, acc.at[pl.ds(off, CHUNK)])

    plsc.subcore_barrier()

    bufs = ((gb0, rb0, sem0), (gb1, rb1, sem1), (gb2, rb2, sem2))

    def _start(k, c):
        gb, rb, sem = bufs[k]
        pltpu.async_copy(y_hbm.at[gidx_v.at[c]], gb, sem)
        pltpu.async_copy(rows_h.at[pl.ds(c, 1)], rb, sem)

    def _finish(k, c):
        gb, rb, sem = bufs[k]
        pltpu.make_async_copy(y_hbm.at[gidx_v.at[c]], gb, sem).wait()
        pltpu.make_async_copy(rows_h.at[pl.ds(c, 1)], rb, sem).wait()
        pltpu.sync_copy(gb, acc.at[rb.at[0]], add=True)

    _start(0, 0)
    _start(1, 1)
    _start(2, 2)

    def _body(j, carry):
        c = 3 * j
        _finish(0, c)
        _start(0, c + 3)
        _finish(1, c + 1)
        _start(1, c + 4)
        _finish(2, c + 2)
        _start(2, c + 5)
        return carry

    lax.fori_loop(0, (NCHUNK - 5) // 3, _body, 0)
    _finish(0, NCHUNK - 5)
    _start(0, NCHUNK - 2)
    _finish(1, NCHUNK - 4)
    _start(1, NCHUNK - 1)
    _finish(2, NCHUNK - 3)
    _finish(0, NCHUNK - 2)
    _finish(1, NCHUNK - 1)

    plsc.subcore_barrier()
    for i in range(pl.cdiv(NROWC, NS)):
        wc = i * NS + sid

        @pl.when(wc < NROWC)
        def _wb_chunk():
            off = pl.multiple_of(wc * CHUNK, CHUNK)
            pltpu.sync_copy(acc.at[pl.ds(off, CHUNK)],
                            out_hbm.at[cid, pl.ds(off, CHUNK)])


def _add_body(p_ref, o_ref):
    o_ref[...] = p_ref[0] + p_ref[1]


def _final_add(partials):
    return pl.pallas_call(
        _add_body,
        grid=(N // _BN,),
        in_specs=[pl.BlockSpec((NC, _BN, C), lambda i: (0, i, 0))],
        out_specs=pl.BlockSpec((_BN, C), lambda i: (i, 0)),
        out_shape=jax.ShapeDtypeStruct((N, C), _f32),
    )(partials)


def kernel(x, edge_index, edge_type, W):
    w3 = W.reshape(T, C, C)
    ei3 = edge_index.reshape(2, _E2, C)
    typ2d = edge_type.reshape(_E2, C)
    y, g2d, r2d = _project(x, w3, ei3, typ2d)
    gidx3 = g2d.reshape(NW, NCHUNK, CHUNK)
    rows3 = r2d.reshape(NW, NCHUNK, CHUNK)
    partials = _sc_aggregate(y.reshape(T * N, C), gidx3, rows3)
    return _final_add(partials)


# speedup vs baseline: 1.0253x; 1.0253x over previous
import functools

import jax
import jax.numpy as jnp
import numpy as np
from jax import lax
from jax.experimental import pallas as pl
from jax.experimental.pallas import tpu as pltpu
from jax.experimental.pallas import tpu_sc as plsc

N = 10000
E = 320000
C = 128
T = 7
NC, NS = 2, 16
NW = NC * NS
EPW = E // NW
CHUNK = 80
NCHUNK = EPW // CHUNK
NROWC = N // CHUNK
SCALE = 1.0 / (T * np.sqrt(float(C)))

_f32 = jnp.float32


_BN = 5000
_E2 = E // C


def _proj_body(x_ref, w_ref, ei_ref, typ_ref, y_ref, g_ref, r_ref):
    x = x_ref[...]
    for t in range(T):
        y_ref[t] = jnp.dot(x, w_ref[t], preferred_element_type=_f32) * SCALE

    @pl.when(pl.program_id(0) == 0)
    def _edge_lists():
        g_ref[...] = typ_ref[...] * N + ei_ref[1]
        r_ref[...] = ei_ref[0]


def _project(x, w3, ei3, typ2d):
    zero3 = lambda nb: (0, 0, 0)
    zero2 = lambda nb: (0, 0)
    return pl.pallas_call(
        _proj_body,
        grid=(N // _BN,),
        in_specs=[
            pl.BlockSpec((_BN, C), lambda nb: (nb, 0)),
            pl.BlockSpec((T, C, C), zero3),
            pl.BlockSpec((2, _E2, C), zero3),
            pl.BlockSpec((_E2, C), zero2),
        ],
        out_specs=[
            pl.BlockSpec((T, _BN, C), lambda nb: (0, nb, 0)),
            pl.BlockSpec((_E2, C), zero2),
            pl.BlockSpec((_E2, C), zero2),
        ],
        out_shape=(
            jax.ShapeDtypeStruct((T, N, C), _f32),
            jax.ShapeDtypeStruct((_E2, C), jnp.int32),
            jax.ShapeDtypeStruct((_E2, C), jnp.int32),
        ),
    )(x, w3, ei3, typ2d)


_mesh = plsc.VectorSubcoreMesh(core_axis_name="c", subcore_axis_name="s")


@functools.partial(
    pl.kernel,
    out_type=jax.ShapeDtypeStruct((NC, N, C), _f32),
    mesh=_mesh,
    scratch_types=[
        pltpu.VMEM((NCHUNK, CHUNK), jnp.int32),
        pltpu.VMEM((1, CHUNK), jnp.int32),
        pltpu.VMEM((1, CHUNK), jnp.int32),
        pltpu.VMEM((1, CHUNK), jnp.int32),
        pltpu.VMEM((CHUNK, C), _f32),
        pltpu.VMEM((CHUNK, C), _f32),
        pltpu.VMEM((CHUNK, C), _f32),
        pltpu.VMEM_SHARED((N, C), _f32),
        pltpu.SemaphoreType.DMA,
        pltpu.SemaphoreType.DMA,
        pltpu.SemaphoreType.DMA,
    ],
    compiler_params=pltpu.CompilerParams(use_tc_tiling_on_sc=False),
)
def _sc_aggregate(y_hbm, gidx_hbm, rows_hbm, out_hbm,
                  gidx_v, rb0, rb1, rb2, gb0, gb1, gb2, acc,
                  sem0, sem1, sem2):
    cid = lax.axis_index("c")
    sid = lax.axis_index("s")
    wid = sid * NC + cid
    rows_h = rows_hbm.at[wid]

    pltpu.sync_copy(gidx_hbm.at[wid], gidx_v)

    zeros16 = jnp.zeros((16,), _f32)

    def _zrow(i, carry):
        for j in range(C // 16):
            gb0[i, pl.ds(j * 16, 16)] = zeros16
        return carry

    lax.fori_loop(0, CHUNK, _zrow, 0)
    for i in range(pl.cdiv(NROWC, NS)):
        zc = i * NS + sid

        @pl.when(zc < NROWC)
        def _zero_chunk():
            off = pl.multiple_of(zc * CHUNK, CHUNK)
            pltpu.sync_copy(gb0, acc.at[pl.ds(off, CHUNK)])

    plsc.subcore_barrier()

    bufs = ((gb0, rb0, sem0), (gb1, rb1, sem1), (gb2, rb2, sem2))

    def _start(k, c):
        gb, rb, sem = bufs[k]
        pltpu.async_copy(y_hbm.at[gidx_v.at[c]], gb, sem)
        pltpu.async_copy(rows_h.at[pl.ds(c, 1)], rb, sem)

    def _finish(k, c):
        gb, rb, sem = bufs[k]
        pltpu.make_async_copy(y_hbm.at[gidx_v.at[c]], gb, sem).wait()
        pltpu.make_async_copy(rows_h.at[pl.ds(c, 1)], rb, sem).wait()
        pltpu.sync_copy(gb, acc.at[rb.at[0]], add=True)

    _start(0, 0)
    _start(1, 1)
    _start(2, 2)

    def _body(j, carry):
        c = 3 * j
        _finish(0, c)
        _start(0, c + 3)
        _finish(1, c + 1)
        _start(1, c + 4)
        _finish(2, c + 2)
        _start(2, c + 5)
        return carry

    lax.fori_loop(0, (NCHUNK - 5) // 3, _body, 0)
    _finish(0, NCHUNK - 5)
    _start(0, NCHUNK - 2)
    _finish(1, NCHUNK - 4)
    _start(1, NCHUNK - 1)
    _finish(2, NCHUNK - 3)
    _finish(0, NCHUNK - 2)
    _finish(1, NCHUNK - 1)

    plsc.subcore_barrier()
    for i in range(pl.cdiv(NROWC, NS)):
        wc = i * NS + sid

        @pl.when(wc < NROWC)
        def _wb_chunk():
            off = pl.multiple_of(wc * CHUNK, CHUNK)
            pltpu.sync_copy(acc.at[pl.ds(off, CHUNK)],
                            out_hbm.at[cid, pl.ds(off, CHUNK)])


def _add_body(p_ref, o_ref):
    o_ref[...] = p_ref[0] + p_ref[1]


def _final_add(partials):
    return pl.pallas_call(
        _add_body,
        grid=(N // _BN,),
        in_specs=[pl.BlockSpec((NC, _BN, C), lambda i: (0, i, 0))],
        out_specs=pl.BlockSpec((_BN, C), lambda i: (i, 0)),
        out_shape=jax.ShapeDtypeStruct((N, C), _f32),
    )(partials)


def kernel(x, edge_index, edge_type, W):
    w3 = W.reshape(T, C, C)
    ei3 = edge_index.reshape(2, _E2, C)
    typ2d = edge_type.reshape(_E2, C)
    y, g2d, r2d = _project(x, w3, ei3, typ2d)
    gidx3 = g2d.reshape(NW, NCHUNK, CHUNK)
    rows3 = r2d.reshape(NW, NCHUNK, CHUNK)
    partials = _sc_aggregate(y.reshape(T * N, C), gidx3, rows3)
    return _final_add(partials)


# speedup vs baseline: 1.0378x; 1.0122x over previous
import functools

import jax
import jax.numpy as jnp
import numpy as np
from jax import lax
from jax.experimental import pallas as pl
from jax.experimental.pallas import tpu as pltpu
from jax.experimental.pallas import tpu_sc as plsc

N = 10000
E = 320000
C = 128
T = 7
NC, NS = 2, 16
NW = NC * NS
EPW = E // NW
CHUNK = 80
NCHUNK = EPW // CHUNK
NROWC = N // CHUNK
SCALE = 1.0 / (T * np.sqrt(float(C)))

_f32 = jnp.float32


_BN = 2000
_E2 = E // C


def _proj_body(x_ref, w_ref, ei_ref, typ_ref, y_ref, g_ref, r_ref):
    x = x_ref[...]
    for t in range(T):
        y_ref[t] = jnp.dot(x, w_ref[t], preferred_element_type=_f32) * SCALE

    @pl.when(pl.program_id(0) == 0)
    def _edge_lists():
        g_ref[...] = typ_ref[...] * N + ei_ref[1]
        r_ref[...] = ei_ref[0]


def _project(x, w3, ei3, typ2d):
    zero3 = lambda nb: (0, 0, 0)
    zero2 = lambda nb: (0, 0)
    return pl.pallas_call(
        _proj_body,
        grid=(N // _BN,),
        in_specs=[
            pl.BlockSpec((_BN, C), lambda nb: (nb, 0)),
            pl.BlockSpec((T, C, C), zero3),
            pl.BlockSpec((2, _E2, C), zero3),
            pl.BlockSpec((_E2, C), zero2),
        ],
        out_specs=[
            pl.BlockSpec((T, _BN, C), lambda nb: (0, nb, 0)),
            pl.BlockSpec((_E2, C), zero2),
            pl.BlockSpec((_E2, C), zero2),
        ],
        out_shape=(
            jax.ShapeDtypeStruct((T, N, C), _f32),
            jax.ShapeDtypeStruct((_E2, C), jnp.int32),
            jax.ShapeDtypeStruct((_E2, C), jnp.int32),
        ),
    )(x, w3, ei3, typ2d)


_mesh = plsc.VectorSubcoreMesh(core_axis_name="c", subcore_axis_name="s")


@functools.partial(
    pl.kernel,
    out_type=jax.ShapeDtypeStruct((NC, N, C), _f32),
    mesh=_mesh,
    scratch_types=[
        pltpu.VMEM((NCHUNK, CHUNK), jnp.int32),
        pltpu.VMEM((1, CHUNK), jnp.int32),
        pltpu.VMEM((1, CHUNK), jnp.int32),
        pltpu.VMEM((1, CHUNK), jnp.int32),
        pltpu.VMEM((CHUNK, C), _f32),
        pltpu.VMEM((CHUNK, C), _f32),
        pltpu.VMEM((CHUNK, C), _f32),
        pltpu.VMEM_SHARED((N, C), _f32),
        pltpu.SemaphoreType.DMA,
        pltpu.SemaphoreType.DMA,
        pltpu.SemaphoreType.DMA,
    ],
    compiler_params=pltpu.CompilerParams(use_tc_tiling_on_sc=False),
)
def _sc_aggregate(y_hbm, gidx_hbm, rows_hbm, out_hbm,
                  gidx_v, rb0, rb1, rb2, gb0, gb1, gb2, acc,
                  sem0, sem1, sem2):
    cid = lax.axis_index("c")
    sid = lax.axis_index("s")
    wid = sid * NC + cid
    rows_h = rows_hbm.at[wid]

    pltpu.sync_copy(gidx_hbm.at[wid], gidx_v)

    zeros16 = jnp.zeros((16,), _f32)

    def _zrow(i, carry):
        for j in range(C // 16):
            gb0[i, pl.ds(j * 16, 16)] = zeros16
        return carry

    lax.fori_loop(0, CHUNK, _zrow, 0)
    for i in range(pl.cdiv(NROWC, NS)):
        zc = i * NS + sid

        @pl.when(zc < NROWC)
        def _zero_chunk():
            off = pl.multiple_of(zc * CHUNK, CHUNK)
            pltpu.sync_copy(gb0, acc.at[pl.ds(off, CHUNK)])

    plsc.subcore_barrier()

    bufs = ((gb0, rb0, sem0), (gb1, rb1, sem1), (gb2, rb2, sem2))

    def _start(k, c):
        gb, rb, sem = bufs[k]
        pltpu.async_copy(y_hbm.at[gidx_v.at[c]], gb, sem)
        pltpu.async_copy(rows_h.at[pl.ds(c, 1)], rb, sem)

    def _finish(k, c):
        gb, rb, sem = bufs[k]
        pltpu.make_async_copy(y_hbm.at[gidx_v.at[c]], gb, sem).wait()
        pltpu.make_async_copy(rows_h.at[pl.ds(c, 1)], rb, sem).wait()
        pltpu.sync_copy(gb, acc.at[rb.at[0]], add=True)

    _start(0, 0)
    _start(1, 1)
    _start(2, 2)

    def _body(j, carry):
        c = 3 * j
        _finish(0, c)
        _start(0, c + 3)
        _finish(1, c + 1)
        _start(1, c + 4)
        _finish(2, c + 2)
        _start(2, c + 5)
        return carry

    lax.fori_loop(0, (NCHUNK - 5) // 3, _body, 0)
    _finish(0, NCHUNK - 5)
    _start(0, NCHUNK - 2)
    _finish(1, NCHUNK - 4)
    _start(1, NCHUNK - 1)
    _finish(2, NCHUNK - 3)
    _finish(0, NCHUNK - 2)
    _finish(1, NCHUNK - 1)

    plsc.subcore_barrier()
    for i in range(pl.cdiv(NROWC, NS)):
        wc = i * NS + sid

        @pl.when(wc < NROWC)
        def _wb_chunk():
            off = pl.multiple_of(wc * CHUNK, CHUNK)
            pltpu.sync_copy(acc.at[pl.ds(off, CHUNK)],
                            out_hbm.at[cid, pl.ds(off, CHUNK)])


def _add_body(p_ref, o_ref):
    o_ref[...] = p_ref[0] + p_ref[1]


def _final_add(partials):
    return pl.pallas_call(
        _add_body,
        grid=(N // _BN,),
        in_specs=[pl.BlockSpec((NC, _BN, C), lambda i: (0, i, 0))],
        out_specs=pl.BlockSpec((_BN, C), lambda i: (i, 0)),
        out_shape=jax.ShapeDtypeStruct((N, C), _f32),
    )(partials)


def kernel(x, edge_index, edge_type, W):
    w3 = W.reshape(T, C, C)
    ei3 = edge_index.reshape(2, _E2, C)
    typ2d = edge_type.reshape(_E2, C)
    y, g2d, r2d = _project(x, w3, ei3, typ2d)
    gidx3 = g2d.reshape(NW, NCHUNK, CHUNK)
    rows3 = r2d.reshape(NW, NCHUNK, CHUNK)
    partials = _sc_aggregate(y.reshape(T * N, C), gidx3, rows3)
    return _final_add(partials)
